# baseline (speedup 1.0000x reference)
import jax
import jax.numpy as jnp
from jax import lax
from jax.experimental import pallas as pl
from jax.experimental.pallas import tpu as pltpu
from jax.experimental.pallas import tpu_sc as plsc

import functools

_NUM_CORES = 2
_NUM_SUBCORES = 16
_NUM_WORKERS = _NUM_CORES * _NUM_SUBCORES

_IDX_MINOR = 128
_GROUP_TOKENS = 128
_NBUF = 4
_LAG = 3
_SG = 8
_TABLE = 256
_PAIR_ROWS = _TABLE * _TABLE


def _make_sc_lookup(total_tokens: int, byte_dim: int):
    pair_dim = 2 * byte_dim
    tokens_per_worker = total_tokens // _NUM_WORKERS
    groups = tokens_per_worker // _GROUP_TOKENS
    idx_rows_per_group = _GROUP_TOKENS // _IDX_MINOR
    his_per_tile = _PAIR_ROWS // _TABLE // _NUM_SUBCORES

    mesh = plsc.VectorSubcoreMesh(core_axis_name="c", subcore_axis_name="s")

    @functools.partial(
        pl.kernel,
        out_type=(
            jax.ShapeDtypeStruct((total_tokens, pair_dim), jnp.float32),
            jax.ShapeDtypeStruct((_NUM_CORES, _PAIR_ROWS, pair_dim), jnp.float32),
        ),
        mesh=mesh,
        scratch_types=[
            pltpu.VMEM((_TABLE // 2, pair_dim), jnp.float32),
            pltpu.VMEM((2, _TABLE // 2, pair_dim), jnp.float32),
            pltpu.VMEM((2, _SG, _IDX_MINOR), jnp.int32),
            pltpu.VMEM((_NBUF, _GROUP_TOKENS, pair_dim), jnp.float32),
            pltpu.SemaphoreType.DMA,
            pltpu.SemaphoreType.DMA,
            pltpu.SemaphoreType.DMA,
        ],
    )
    def sc_lookup(tokens_hbm, w2_hbm, out_hbm, wq_hbm, w2_v, block_v, idx_v,
                  rows_v, gsem, wsem, isem):
        cid = lax.axis_index("c")
        sid = lax.axis_index("s")
        wid = sid * _NUM_CORES + cid
        n_vecs = byte_dim // 16
        half = _TABLE // 2

        def w_row_vecs(j):
            off = (j & 1) * byte_dim
            return [w2_v[j >> 1, pl.ds(off + i * 16, 16)] for i in range(n_vecs)]

        tok_row_base = wid * (tokens_per_worker // _IDX_MINOR)
        idesc0 = pltpu.async_copy(tokens_hbm.at[pl.ds(tok_row_base, _SG)],
                                  idx_v.at[0], isem)
        pltpu.sync_copy(w2_hbm, w2_v)

        def left_body(l, carry):
            for h in range(2):
                regs = w_row_vecs(h * half + l)
                for i in range(n_vecs):
                    block_v[h, l, pl.ds(i * 16, 16)] = regs[i]
            return carry

        lax.fori_loop(0, half, left_body, 0, unroll=4)

        bdesc = {}
        for b in range(his_per_tile):
            hi = sid * his_per_tile + b
            row_regs = w_row_vecs(hi)
            for h in range(2):
                if b > 0:
                    bdesc[h].wait()

                def row_body(k, carry, regs=row_regs, h=h):
                    for i in range(n_vecs):
                        block_v[h, k, pl.ds(byte_dim + i * 16, 16)] = regs[i]
                    return carry

                lax.fori_loop(0, half, row_body, 0, unroll=8)
                bdesc[h] = pltpu.async_copy(
                    block_v.at[h],
                    wq_hbm.at[cid, pl.ds(hi * _TABLE + h * half, half)], wsem)
        bdesc[0].wait()
        bdesc[1].wait()
        plsc.subcore_barrier()

        out_base = wid * tokens_per_worker
        n_supergroups = groups // _SG
        n_groups = n_supergroups * _SG

        def fire_gather(g, sp, b):
            return pltpu.async_copy(
                wq_hbm.at[cid].at[idx_v.at[sp, b]],
                rows_v.at[g % _NBUF], gsem)

        def fire_write(g):
            return pltpu.async_copy(
                rows_v.at[g % _NBUF],
                out_hbm.at[pl.ds(out_base + g * _GROUP_TOKENS, _GROUP_TOKENS)],
                wsem)

        gdesc, wdesc = {}, {}
        idesc = {}
        idesc0.wait()
        for s in range(n_supergroups):
            sp = s & 1
            if s > 0:
                idesc[s].wait()
            for b in range(_SG):
                g = s * _SG + b
                if g >= _NBUF:
                    wdesc[g - _NBUF].wait()
                gdesc[g] = fire_gather(g, sp, b)
                if b == 3 and s + 1 < n_supergroups:
                    idesc[s + 1] = pltpu.async_copy(
                        tokens_hbm.at[pl.ds(tok_row_base + (s + 1) * _SG, _SG)],
                        idx_v.at[1 - sp], isem)
                if g >= _LAG:
                    gdesc[g - _LAG].wait()
                    wdesc[g - _LAG] = fire_write(g - _LAG)
        for g in range(n_groups - _LAG, n_groups):
            gdesc[g].wait()
            wdesc[g] = fire_write(g)
        for g in range(n_groups - _NBUF, n_groups):
            wdesc[g].wait()

    return sc_lookup


def kernel(tokens, W):
    batch, seq = tokens.shape
    byte_dim = W.shape[1]
    total_tokens = batch * seq
    tokens2 = tokens.reshape(total_tokens // _IDX_MINOR, _IDX_MINOR)
    w2 = W.reshape(W.shape[0] // 2, 2 * byte_dim)
    rows, _ = _make_sc_lookup(total_tokens, byte_dim)(tokens2, w2)
    return rows.reshape(batch, seq, 2 * byte_dim)
